# baseline (speedup 1.0000x reference)
import functools

import jax
import jax.numpy as jnp
from jax.experimental import pallas as pl
from jax.experimental.pallas import tpu as pltpu

_LATENT_DIM = 64
_NUM_CODES = 1024
_ROWS = 8192
_SUB_ROWS = 256
_N_CHUNKS = _ROWS // _SUB_ROWS


def _vq_body(x_hbm, emb_ref, o_hbm, xbuf, obuf, insem, outsem):
    emb = emb_ref[...]
    e2 = jnp.sum(emb * emb, axis=0, keepdims=True)

    def in_copy(k, slot):
        return pltpu.make_async_copy(
            x_hbm.at[pl.ds(k * _SUB_ROWS, _SUB_ROWS), :],
            xbuf.at[slot], insem.at[slot])

    def out_copy(k, slot):
        return pltpu.make_async_copy(
            obuf.at[slot],
            o_hbm.at[pl.ds(k * _SUB_ROWS, _SUB_ROWS), :], outsem.at[slot])

    in_copy(0, 0).start()
    for k in range(_N_CHUNKS):
        slot = k % 2
        if k + 1 < _N_CHUNKS:
            in_copy(k + 1, (k + 1) % 2).start()
        in_copy(k, slot).wait()
        xb = xbuf[slot]
        sim = jnp.dot(xb, emb, preferred_element_type=jnp.float32)
        scores = e2 - 2.0 * sim
        idx = jnp.argmin(scores, axis=1).reshape(-1, 1)
        col = jax.lax.broadcasted_iota(jnp.int32, scores.shape, 1)
        onehot = (col == idx).astype(jnp.float32)
        if k >= 2:
            out_copy(k - 2, slot).wait()
        obuf[slot] = jax.lax.dot_general(
            onehot, emb, (((1,), (1,)), ((), ())),
            preferred_element_type=jnp.float32)
        out_copy(k, slot).start()
    out_copy(_N_CHUNKS - 2, (_N_CHUNKS - 2) % 2).wait()
    out_copy(_N_CHUNKS - 1, (_N_CHUNKS - 1) % 2).wait()


@functools.partial(jax.jit, static_argnames=("interpret",))
def kernel(x, embeddings, interpret=False):
    orig_shape = x.shape
    xf = x.reshape(-1, _LATENT_DIM)
    out = pl.pallas_call(
        _vq_body,
        in_specs=[
            pl.BlockSpec(memory_space=pltpu.MemorySpace.HBM),
            pl.BlockSpec(memory_space=pltpu.MemorySpace.VMEM),
        ],
        out_specs=pl.BlockSpec(memory_space=pltpu.MemorySpace.HBM),
        out_shape=jax.ShapeDtypeStruct((_ROWS, _LATENT_DIM), jnp.float32),
        scratch_shapes=[
            pltpu.MemorySpace.VMEM((2, _SUB_ROWS, _LATENT_DIM), jnp.float32),
            pltpu.MemorySpace.VMEM((2, _SUB_ROWS, _LATENT_DIM), jnp.float32),
            pltpu.SemaphoreType.DMA((2,)),
            pltpu.SemaphoreType.DMA((2,)),
        ],
        interpret=interpret,
    )(xf, embeddings)
    return out.reshape(orig_shape)


# speedup vs baseline: 1.9091x; 1.9091x over previous
import functools

import jax
import jax.numpy as jnp
from jax.experimental import pallas as pl
from jax.experimental.pallas import tpu as pltpu

_LATENT_DIM = 64
_NUM_CODES = 1024
_ROWS = 8192
_PIECE_ROWS = 4096
_N_PIECES = _ROWS // _PIECE_ROWS
_SUB_ROWS = 256
_SUBS_PER_PIECE = _PIECE_ROWS // _SUB_ROWS
_N_CHUNKS = _ROWS // _SUB_ROWS


def _vq_body(x_hbm, emb_ref, o_hbm, xbuf, obuf, insem, outsem):
    emb = emb_ref[...]
    e2 = jnp.sum(emb * emb, axis=0, keepdims=True)

    def in_copy(p):
        return pltpu.make_async_copy(
            x_hbm.at[pl.ds(p * _PIECE_ROWS, _PIECE_ROWS), :],
            xbuf.at[p % 2], insem.at[p % 2])

    def out_copy(k):
        return pltpu.make_async_copy(
            obuf.at[k],
            o_hbm.at[pl.ds(k * _SUB_ROWS, _SUB_ROWS), :], outsem.at[k])

    in_copy(0).start()
    for p in range(_N_PIECES):
        if p + 1 < _N_PIECES:
            in_copy(p + 1).start()
        in_copy(p).wait()
        for j in range(_SUBS_PER_PIECE):
            k = p * _SUBS_PER_PIECE + j
            xb = xbuf[p % 2, pl.ds(j * _SUB_ROWS, _SUB_ROWS), :]
            sim = jnp.dot(xb, emb, preferred_element_type=jnp.float32)
            scores = e2 - 2.0 * sim
            idx = jnp.argmin(scores, axis=1).reshape(-1, 1)
            col = jax.lax.broadcasted_iota(jnp.int32, scores.shape, 1)
            onehot = (col == idx).astype(jnp.float32)
            obuf[k] = jax.lax.dot_general(
                onehot, emb, (((1,), (1,)), ((), ())),
                preferred_element_type=jnp.float32)
            out_copy(k).start()
    for k in range(_N_CHUNKS):
        out_copy(k).wait()


@functools.partial(jax.jit, static_argnames=("interpret",))
def kernel(x, embeddings, interpret=False):
    orig_shape = x.shape
    xf = x.reshape(-1, _LATENT_DIM)
    out = pl.pallas_call(
        _vq_body,
        in_specs=[
            pl.BlockSpec(memory_space=pltpu.MemorySpace.HBM),
            pl.BlockSpec(memory_space=pltpu.MemorySpace.VMEM),
        ],
        out_specs=pl.BlockSpec(memory_space=pltpu.MemorySpace.HBM),
        out_shape=jax.ShapeDtypeStruct((_ROWS, _LATENT_DIM), jnp.float32),
        scratch_shapes=[
            pltpu.MemorySpace.VMEM((2, _PIECE_ROWS, _LATENT_DIM), jnp.float32),
            pltpu.MemorySpace.VMEM((_N_CHUNKS, _SUB_ROWS, _LATENT_DIM),
                                   jnp.float32),
            pltpu.SemaphoreType.DMA((2,)),
            pltpu.SemaphoreType.DMA((_N_CHUNKS,)),
        ],
        interpret=interpret,
    )(xf, embeddings)
    return out.reshape(orig_shape)
